# baseline (speedup 1.0000x reference)
import functools

import jax
import jax.numpy as jnp
from jax import lax
from jax.experimental import pallas as pl
from jax.experimental.pallas import tpu as pltpu
from jax.experimental.pallas import tpu_sc as plsc

NG = 20000
NGP = 20480
F = 32
K = 8
QTOT = 32768
TW = 48

BQ = 128
BQ3 = 1024


def _topk_body(x_ref, mt_ref, idx_ref):
    xb = x_ref[...]
    mt = mt_ref[...]
    msq = jnp.sum(mt * mt, axis=0, keepdims=True)
    scores = msq - 2.0 * jnp.dot(xb, mt, preferred_element_type=jnp.float32)
    iota = lax.broadcasted_iota(jnp.int32, (BQ, NGP), 1)
    cols = []
    for _ in range(K):
        rowmin = jnp.min(scores, axis=1, keepdims=True)
        cand = jnp.where(scores <= rowmin, iota, jnp.int32(2**30))
        idxj = jnp.min(cand, axis=1, keepdims=True)
        cols.append(idxj)
        scores = jnp.where(iota == idxj, jnp.inf, scores)
    idx_ref[...] = jnp.concatenate(cols, axis=1)


def _topk(xp, mt):
    return pl.pallas_call(
        _topk_body,
        grid=(QTOT // BQ,),
        in_specs=[
            pl.BlockSpec((BQ, 8), lambda i: (i, 0)),
            pl.BlockSpec((8, NGP), lambda i: (0, 0)),
        ],
        out_specs=pl.BlockSpec((BQ, K), lambda i: (i, 0)),
        out_shape=jax.ShapeDtypeStruct((QTOT, K), jnp.int32),
        compiler_params=pltpu.CompilerParams(
            dimension_semantics=("parallel",),
        ),
    )(xp, mt)


NW = 32
ROWS = QTOT * K
RPW = ROWS // NW
CH = 128
NCH = RPW // CH


def _gather_body(table_hbm, idx_hbm, out_hbm, idx_v, buf0, buf1, sem0, sem1):
    wid = lax.axis_index("s") * 2 + lax.axis_index("c")
    base = wid * RPW
    pltpu.sync_copy(idx_hbm.at[pl.ds(base, RPW)], idx_v)

    def _start(c, buf, sem):
        pltpu.make_async_copy(
            table_hbm.at[idx_v.at[pl.ds(c * CH, CH)]], buf, sem).start()

    def _wait(c, buf, sem):
        pltpu.make_async_copy(
            table_hbm.at[idx_v.at[pl.ds(c * CH, CH)]], buf, sem).wait()

    _start(0, buf0, sem0)

    def body(i, carry):
        c0 = 2 * i
        _start(c0 + 1, buf1, sem1)
        _wait(c0, buf0, sem0)
        pltpu.sync_copy(buf0, out_hbm.at[pl.ds(base + c0 * CH, CH)])

        @pl.when(c0 + 2 < NCH)
        def _():
            _start(c0 + 2, buf0, sem0)

        _wait(c0 + 1, buf1, sem1)
        pltpu.sync_copy(buf1, out_hbm.at[pl.ds(base + (c0 + 1) * CH, CH)])
        return carry

    lax.fori_loop(0, NCH // 2, body, 0)


@functools.cache
def _gather_k():
    return pl.kernel(
        _gather_body,
        out_type=jax.ShapeDtypeStruct((ROWS, TW), jnp.float32),
        mesh=plsc.VectorSubcoreMesh(core_axis_name="c", subcore_axis_name="s"),
        scratch_types=[
            pltpu.VMEM((RPW,), jnp.int32),
            pltpu.VMEM((CH, TW), jnp.float32),
            pltpu.VMEM((CH, TW), jnp.float32),
            pltpu.SemaphoreType.DMA,
            pltpu.SemaphoreType.DMA,
        ],
        compiler_params=pltpu.CompilerParams(use_tc_tiling_on_sc=False),
    )


def _wsum_body(x_ref, g_ref, out_ref):
    xb = x_ref[...][:, :3]
    g = g_ref[...]
    fk = g[..., 0:F]
    mk = g[..., F:F + 3]
    lk = g[..., F + 3:F + 6]
    diff = xb[:, None, :] - mk
    logw = -0.5 * jnp.sum(diff * diff / jnp.exp(lk), axis=-1)
    logw = logw - jnp.max(logw, axis=-1, keepdims=True)
    w = jnp.exp(logw)
    w = w / (jnp.sum(w, axis=-1, keepdims=True) + 1e-8)
    out_ref[...] = jnp.sum(w[..., None] * fk, axis=1)


def _wsum(xp, g3):
    return pl.pallas_call(
        _wsum_body,
        grid=(QTOT // BQ3,),
        in_specs=[
            pl.BlockSpec((BQ3, 8), lambda i: (i, 0)),
            pl.BlockSpec((BQ3, K, TW), lambda i: (i, 0, 0)),
        ],
        out_specs=pl.BlockSpec((BQ3, F), lambda i: (i, 0)),
        out_shape=jax.ShapeDtypeStruct((QTOT, F), jnp.float32),
        compiler_params=pltpu.CompilerParams(
            dimension_semantics=("parallel",),
        ),
    )(xp, g3)


def kernel(x, means, feats, log_covs):
    xp = jnp.pad(x, ((0, 0), (0, 5)))
    mpad = jnp.pad(means, ((0, NGP - NG), (0, 0)), constant_values=1000.0)
    mt = jnp.pad(mpad.T, ((0, 5), (0, 0)))
    table = jnp.concatenate(
        [feats, means, log_covs, jnp.zeros((NG, TW - F - 6), jnp.float32)],
        axis=1)

    idx = _topk(xp, mt)
    gath = _gather_k()(table, idx.reshape(-1))
    return _wsum(xp, gath.reshape(QTOT, K, TW))


# speedup vs baseline: 1.4255x; 1.4255x over previous
import functools

import jax
import jax.numpy as jnp
from jax import lax
from jax.experimental import pallas as pl
from jax.experimental.pallas import tpu as pltpu
from jax.experimental.pallas import tpu_sc as plsc

NG = 20000
NGP = 20480
F = 32
K = 8
QTOT = 32768
TW = 48

BQ = 128
BQ3 = 1024


_SORT8 = [(0, 1), (2, 3), (4, 5), (6, 7),
          (0, 2), (1, 3), (4, 6), (5, 7),
          (1, 2), (5, 6),
          (0, 4), (1, 5), (2, 6), (3, 7),
          (2, 4), (3, 5),
          (1, 2), (3, 4), (5, 6)]

_BITONIC8 = [(0, 4), (1, 5), (2, 6), (3, 7),
             (0, 2), (1, 3), (4, 6), (5, 7),
             (0, 1), (2, 3), (4, 5), (6, 7)]


def _ce(a, b):
    va, ia = a
    vb, ib = b
    cond = va <= vb
    return ((jnp.minimum(va, vb), jnp.where(cond, ia, ib)),
            (jnp.maximum(va, vb), jnp.where(cond, ib, ia)))


def _merge_pair(A, B):
    C = []
    for e in range(K):
        va, ia = A[e]
        vb, ib = B[K - 1 - e]
        cond = va <= vb
        C.append((jnp.minimum(va, vb), jnp.where(cond, ia, ib)))
    for p, q in _BITONIC8:
        C[p], C[q] = _ce(C[p], C[q])
    return C


def _topk_body(x_ref, mt_ref, idx_ref):
    xb = x_ref[...]
    mt = mt_ref[...]
    msq = jnp.sum(mt * mt, axis=0, keepdims=True)
    scores = msq - 2.0 * jnp.dot(xb, mt, preferred_element_type=jnp.float32)

    CW = NGP // K
    iota = lax.broadcasted_iota(jnp.int32, (BQ, CW), 1)
    planes = [(scores[:, e * CW:(e + 1) * CW], iota + e * CW)
              for e in range(K)]
    for p, q in _SORT8:
        planes[p], planes[q] = _ce(planes[p], planes[q])

    pending = []
    w = CW
    while w > 1:
        w2 = w // 2
        if w % 2 == 1:
            pending.append([(v[:, w - 1:w], i[:, w - 1:w]) for v, i in planes])
        A = [(v[:, :w2], i[:, :w2]) for v, i in planes]
        B = [(v[:, w2:2 * w2], i[:, w2:2 * w2]) for v, i in planes]
        planes = _merge_pair(A, B)
        w = w2
    for P in pending:
        planes = _merge_pair(planes, P)

    idx_ref[...] = jnp.concatenate([i for _, i in planes], axis=1)


def _topk(xp, mt):
    return pl.pallas_call(
        _topk_body,
        grid=(QTOT // BQ,),
        in_specs=[
            pl.BlockSpec((BQ, 8), lambda i: (i, 0)),
            pl.BlockSpec((8, NGP), lambda i: (0, 0)),
        ],
        out_specs=pl.BlockSpec((BQ, K), lambda i: (i, 0)),
        out_shape=jax.ShapeDtypeStruct((QTOT, K), jnp.int32),
        compiler_params=pltpu.CompilerParams(
            dimension_semantics=("parallel",),
        ),
    )(xp, mt)


NW = 32
ROWS = QTOT * K
RPW = ROWS // NW
CH = 128
NCH = RPW // CH


def _gather_body(table_hbm, idx_hbm, out_hbm, idx_v, buf0, buf1, sem0, sem1):
    wid = lax.axis_index("s") * 2 + lax.axis_index("c")
    base = wid * RPW
    pltpu.sync_copy(idx_hbm.at[pl.ds(base, RPW)], idx_v)

    def _start(c, buf, sem):
        pltpu.make_async_copy(
            table_hbm.at[idx_v.at[pl.ds(c * CH, CH)]], buf, sem).start()

    def _wait(c, buf, sem):
        pltpu.make_async_copy(
            table_hbm.at[idx_v.at[pl.ds(c * CH, CH)]], buf, sem).wait()

    _start(0, buf0, sem0)

    def body(i, carry):
        c0 = 2 * i
        _start(c0 + 1, buf1, sem1)
        _wait(c0, buf0, sem0)
        pltpu.sync_copy(buf0, out_hbm.at[pl.ds(base + c0 * CH, CH)])

        @pl.when(c0 + 2 < NCH)
        def _():
            _start(c0 + 2, buf0, sem0)

        _wait(c0 + 1, buf1, sem1)
        pltpu.sync_copy(buf1, out_hbm.at[pl.ds(base + (c0 + 1) * CH, CH)])
        return carry

    lax.fori_loop(0, NCH // 2, body, 0)


@functools.cache
def _gather_k():
    return pl.kernel(
        _gather_body,
        out_type=jax.ShapeDtypeStruct((ROWS, TW), jnp.float32),
        mesh=plsc.VectorSubcoreMesh(core_axis_name="c", subcore_axis_name="s"),
        scratch_types=[
            pltpu.VMEM((RPW,), jnp.int32),
            pltpu.VMEM((CH, TW), jnp.float32),
            pltpu.VMEM((CH, TW), jnp.float32),
            pltpu.SemaphoreType.DMA,
            pltpu.SemaphoreType.DMA,
        ],
        compiler_params=pltpu.CompilerParams(use_tc_tiling_on_sc=False),
    )


def _wsum_body(x_ref, g_ref, out_ref):
    xb = x_ref[...][:, :3]
    g = g_ref[...]
    fk = g[..., 0:F]
    mk = g[..., F:F + 3]
    lk = g[..., F + 3:F + 6]
    diff = xb[:, None, :] - mk
    logw = -0.5 * jnp.sum(diff * diff / jnp.exp(lk), axis=-1)
    logw = logw - jnp.max(logw, axis=-1, keepdims=True)
    w = jnp.exp(logw)
    w = w / (jnp.sum(w, axis=-1, keepdims=True) + 1e-8)
    out_ref[...] = jnp.sum(w[..., None] * fk, axis=1)


def _wsum(xp, g3):
    return pl.pallas_call(
        _wsum_body,
        grid=(QTOT // BQ3,),
        in_specs=[
            pl.BlockSpec((BQ3, 8), lambda i: (i, 0)),
            pl.BlockSpec((BQ3, K, TW), lambda i: (i, 0, 0)),
        ],
        out_specs=pl.BlockSpec((BQ3, F), lambda i: (i, 0)),
        out_shape=jax.ShapeDtypeStruct((QTOT, F), jnp.float32),
        compiler_params=pltpu.CompilerParams(
            dimension_semantics=("parallel",),
        ),
    )(xp, g3)


def kernel(x, means, feats, log_covs):
    xp = jnp.pad(x, ((0, 0), (0, 5)))
    mpad = jnp.pad(means, ((0, NGP - NG), (0, 0)), constant_values=1000.0)
    mt = jnp.pad(mpad.T, ((0, 5), (0, 0)))
    table = jnp.concatenate(
        [feats, means, log_covs, jnp.zeros((NG, TW - F - 6), jnp.float32)],
        axis=1)

    idx = _topk(xp, mt)
    gath = _gather_k()(table, idx.reshape(-1))
    return _wsum(xp, gath.reshape(QTOT, K, TW))


# speedup vs baseline: 1.4373x; 1.0083x over previous
import functools

import jax
import jax.numpy as jnp
from jax import lax
from jax.experimental import pallas as pl
from jax.experimental.pallas import tpu as pltpu
from jax.experimental.pallas import tpu_sc as plsc

NG = 20000
NGP = 20480
F = 32
K = 8
QTOT = 32768
TW = 48

BQ = 128
BQ3 = 1024


_SORT8 = [(0, 1), (2, 3), (4, 5), (6, 7),
          (0, 2), (1, 3), (4, 6), (5, 7),
          (1, 2), (5, 6),
          (0, 4), (1, 5), (2, 6), (3, 7),
          (2, 4), (3, 5),
          (1, 2), (3, 4), (5, 6)]

_BITONIC8 = [(0, 4), (1, 5), (2, 6), (3, 7),
             (0, 2), (1, 3), (4, 6), (5, 7),
             (0, 1), (2, 3), (4, 5), (6, 7)]


def _ce(a, b):
    va, ia = a
    vb, ib = b
    cond = va <= vb
    return ((jnp.minimum(va, vb), jnp.where(cond, ia, ib)),
            (jnp.maximum(va, vb), jnp.where(cond, ib, ia)))


def _merge_pair(A, B):
    C = []
    for e in range(K):
        va, ia = A[e]
        vb, ib = B[K - 1 - e]
        cond = va <= vb
        C.append((jnp.minimum(va, vb), jnp.where(cond, ia, ib)))
    for p, q in _BITONIC8:
        C[p], C[q] = _ce(C[p], C[q])
    return C


def _topk_body(x_ref, mt_ref, idx_ref):
    xb = x_ref[...]
    mt = mt_ref[...]
    msq = jnp.sum(mt * mt, axis=0, keepdims=True)
    scores = msq + jnp.dot(xb, mt, preferred_element_type=jnp.float32)

    CW = NGP // K
    iota = lax.broadcasted_iota(jnp.int32, (BQ, CW), 1)
    planes = [(scores[:, e * CW:(e + 1) * CW], iota + e * CW)
              for e in range(K)]
    for p, q in _SORT8:
        planes[p], planes[q] = _ce(planes[p], planes[q])

    pending = []
    w = CW
    while w > 1:
        w2 = w // 2
        if w % 2 == 1:
            pending.append([(v[:, w - 1:w], i[:, w - 1:w]) for v, i in planes])
        A = [(v[:, :w2], i[:, :w2]) for v, i in planes]
        B = [(v[:, w2:2 * w2], i[:, w2:2 * w2]) for v, i in planes]
        planes = _merge_pair(A, B)
        w = w2
    for P in pending:
        planes = _merge_pair(planes, P)

    idx_ref[...] = jnp.concatenate([i for _, i in planes], axis=1)


def _topk(xa, mt):
    q = xa.shape[0]
    return pl.pallas_call(
        _topk_body,
        grid=(q // BQ,),
        in_specs=[
            pl.BlockSpec((BQ, 8), lambda i: (i, 0)),
            pl.BlockSpec((8, NGP), lambda i: (0, 0)),
        ],
        out_specs=pl.BlockSpec((BQ, K), lambda i: (i, 0)),
        out_shape=jax.ShapeDtypeStruct((q, K), jnp.int32),
        compiler_params=pltpu.CompilerParams(
            dimension_semantics=("parallel",),
        ),
    )(xa, mt)


NW = 32
ROWS = QTOT * K
RPW = ROWS // NW
CH = 128
NCH = RPW // CH


def _make_gather_body(rpw, nch):
    def _gather_body(table_hbm, idx_hbm, out_hbm, idx_v, buf0, buf1,
                     sem0, sem1):
        wid = lax.axis_index("s") * 2 + lax.axis_index("c")
        base = wid * rpw
        pltpu.sync_copy(idx_hbm.at[pl.ds(base, rpw)], idx_v)

        def _start(c, buf, sem):
            pltpu.make_async_copy(
                table_hbm.at[idx_v.at[pl.ds(c * CH, CH)]], buf, sem).start()

        def _wait(c, buf, sem):
            pltpu.make_async_copy(
                table_hbm.at[idx_v.at[pl.ds(c * CH, CH)]], buf, sem).wait()

        _start(0, buf0, sem0)

        def body(i, carry):
            c0 = 2 * i
            _start(c0 + 1, buf1, sem1)
            _wait(c0, buf0, sem0)
            pltpu.sync_copy(buf0, out_hbm.at[pl.ds(base + c0 * CH, CH)])

            @pl.when(c0 + 2 < nch)
            def _():
                _start(c0 + 2, buf0, sem0)

            _wait(c0 + 1, buf1, sem1)
            pltpu.sync_copy(buf1, out_hbm.at[pl.ds(base + (c0 + 1) * CH, CH)])
            return carry

        lax.fori_loop(0, nch // 2, body, 0)

    return _gather_body


@functools.cache
def _gather_k(rows):
    rpw = rows // NW
    nch = rpw // CH
    return pl.kernel(
        _make_gather_body(rpw, nch),
        out_type=jax.ShapeDtypeStruct((rows, TW), jnp.float32),
        mesh=plsc.VectorSubcoreMesh(core_axis_name="c", subcore_axis_name="s"),
        scratch_types=[
            pltpu.VMEM((rpw,), jnp.int32),
            pltpu.VMEM((CH, TW), jnp.float32),
            pltpu.VMEM((CH, TW), jnp.float32),
            pltpu.SemaphoreType.DMA,
            pltpu.SemaphoreType.DMA,
        ],
        compiler_params=pltpu.CompilerParams(use_tc_tiling_on_sc=False),
    )


def _wsum_body(x_ref, g_ref, out_ref):
    xb = x_ref[...][:, :3]
    g = g_ref[...]
    fk = g[..., 0:F]
    mk = g[..., F:F + 3]
    lk = g[..., F + 3:F + 6]
    diff = xb[:, None, :] - mk
    logw = -0.5 * jnp.sum(diff * diff / jnp.exp(lk), axis=-1)
    logw = logw - jnp.max(logw, axis=-1, keepdims=True)
    w = jnp.exp(logw)
    w = w / (jnp.sum(w, axis=-1, keepdims=True) + 1e-8)
    out_ref[...] = jnp.sum(w[..., None] * fk, axis=1)


def _wsum(xp, g3):
    return pl.pallas_call(
        _wsum_body,
        grid=(QTOT // BQ3,),
        in_specs=[
            pl.BlockSpec((BQ3, 8), lambda i: (i, 0)),
            pl.BlockSpec((BQ3, K, TW), lambda i: (i, 0, 0)),
        ],
        out_specs=pl.BlockSpec((BQ3, F), lambda i: (i, 0)),
        out_shape=jax.ShapeDtypeStruct((QTOT, F), jnp.float32),
        compiler_params=pltpu.CompilerParams(
            dimension_semantics=("parallel",),
        ),
    )(xp, g3)


def kernel(x, means, feats, log_covs):
    xp = jnp.pad(x, ((0, 0), (0, 5)))
    xa = jnp.pad(-2.0 * x, ((0, 0), (0, 5)))
    mpad = jnp.pad(means, ((0, NGP - NG), (0, 0)), constant_values=1000.0)
    mt = jnp.pad(mpad.T, ((0, 5), (0, 0)))
    table = jnp.concatenate(
        [feats, means, log_covs, jnp.zeros((NG, TW - F - 6), jnp.float32)],
        axis=1)

    half = QTOT // 2
    gs = []
    for h in range(2):
        idx = _topk(xa[h * half:(h + 1) * half], mt)
        gs.append(_gather_k(half * K)(table, idx.reshape(-1)))
    gath = jnp.concatenate(gs, axis=0)
    return _wsum(xp, gath.reshape(QTOT, K, TW))
